# Initial kernel scaffold; baseline (speedup 1.0000x reference)
import jax
import jax.numpy as jnp
from jax.experimental import pallas as pl


def kernel(seq_pitch_type, seq_cont, seq_swing_attempt, seq_swing_result, pt_table, sr_table):
    raise NotImplementedError("write your pallas kernel here")


# speedup vs baseline: 3.0987x; 3.0987x over previous
import functools

import jax
import jax.numpy as jnp
from jax import lax
from jax.experimental import pallas as pl
from jax.experimental.pallas import tpu as pltpu
from jax.experimental.pallas import tpu_sc as plsc

NUM_PT = 1000
PT_D = 32
NUM_SR = 8
SR_D = 4
NUM_CONT = 16
OUT_D = PT_D + NUM_CONT + 1 + SR_D
TAIL_W = 8
ATT_SLOT = 3

NC, NS, LANES = 2, 16, 16
NW = NC * NS
T = 1024
IDX_W = 128

TC_BLK = 4096


def _sc_gather(N, pt_idx2, sr_idxT, att, pt_table, pair_tab):
    per_w = N // NW
    chunks = per_w // T
    mesh = plsc.VectorSubcoreMesh(core_axis_name="c", subcore_axis_name="s")

    @functools.partial(
        pl.kernel,
        mesh=mesh,
        compiler_params=pltpu.CompilerParams(use_tc_tiling_on_sc=False),
        out_type=(jax.ShapeDtypeStruct((N, PT_D), jnp.float32),
                  jax.ShapeDtypeStruct((N // 2, LANES), jnp.float32)),
        scratch_types=[
            pltpu.VMEM((T // IDX_W, IDX_W), jnp.int32),
            pltpu.VMEM((2, T // 2), jnp.int32),
            pltpu.VMEM((T // 2 // IDX_W, IDX_W), jnp.int32),
            pltpu.VMEM((T + LANES,), jnp.float32),
            pltpu.VMEM((T, PT_D), jnp.float32),
            pltpu.VMEM((T // 2, LANES), jnp.float32),
            pltpu.SemaphoreType.DMA,
            pltpu.SemaphoreType.DMA,
        ],
    )
    def run(pt_idx2_hbm, sr_idxT_hbm, att_hbm, pt_tab_hbm, pair_tab_hbm,
            ptg_hbm, tail_hbm, idx_v, sridx_v, pair_v, att_v, rows_v,
            tails_v, sem0, sem1):
        wid = lax.axis_index("s") * NC + lax.axis_index("c")
        base_w = wid * per_w

        lane = lax.iota(jnp.int32, LANES)
        att_lane = (lane & (TAIL_W - 1)) == ATT_SLOT

        def chunk_body(i, _):
            base = pl.multiple_of(base_w + i * T, T)
            r0 = pl.multiple_of(base // IDX_W, T // IDX_W)
            pltpu.sync_copy(pt_idx2_hbm.at[pl.ds(r0, T // IDX_W)], idx_v)
            h0 = pl.multiple_of(base // 2, T // 2)
            pltpu.sync_copy(sr_idxT_hbm.at[:, pl.ds(h0, T // 2)], sridx_v)
            pltpu.sync_copy(att_hbm.at[pl.ds(base, T)], att_v.at[pl.ds(0, T)])

            for r in range(T // IDX_W):
                for k in range(IDX_W // LANES):
                    sl = pl.ds(k * LANES, LANES)
                    idx_v[r, sl] = jnp.clip(idx_v[r, sl], 0, NUM_PT)
            for r in range(T // 2 // IDX_W):
                for k in range(IDX_W // LANES):
                    sl = pl.ds(r * IDX_W + k * LANES, LANES)
                    s0 = sridx_v[0, sl]
                    s0 = jnp.where(s0 < 0, NUM_SR, s0)
                    s1 = sridx_v[1, sl]
                    s1 = jnp.where(s1 < 0, NUM_SR, s1)
                    pair_v[r, pl.ds(k * LANES, LANES)] = (
                        s0 * (NUM_SR + 1) + s1)

            copies = []
            for r in range(T // IDX_W):
                copies.append(pltpu.async_copy(
                    pt_tab_hbm.at[idx_v.at[r]],
                    rows_v.at[pl.ds(r * IDX_W, IDX_W)], sem0))
            for r in range(T // 2 // IDX_W):
                copies.append(pltpu.async_copy(
                    pair_tab_hbm.at[pair_v.at[r]],
                    tails_v.at[pl.ds(r * IDX_W, IDX_W)], sem1))
            for c in copies:
                c.wait()

            for p in range(T // 2):
                av = att_v[pl.ds(2 * p, LANES)]
                a = jnp.where(lane < TAIL_W, av[0], av[1])
                tails_v[p] = jnp.where(att_lane, a, tails_v[p])

            pltpu.sync_copy(rows_v, ptg_hbm.at[pl.ds(base, T)])
            pltpu.sync_copy(tails_v, tail_hbm.at[pl.ds(base // 2, T // 2)])
            return ()

        lax.fori_loop(0, chunks, chunk_body, (), unroll=False)

    return run(pt_idx2, sr_idxT, att, pt_table, pair_tab)


def _tc_concat(N, ptg, cont, tail8):
    grid = (N // TC_BLK,)

    def body(pt_ref, cont_ref, tail_ref, out_ref):
        out_ref[...] = jnp.concatenate(
            [pt_ref[...], cont_ref[...],
             tail_ref[:, ATT_SLOT:TAIL_W]], axis=1)

    return pl.pallas_call(
        body,
        grid=grid,
        in_specs=[
            pl.BlockSpec((TC_BLK, PT_D), lambda i: (i, 0)),
            pl.BlockSpec((TC_BLK, NUM_CONT), lambda i: (i, 0)),
            pl.BlockSpec((TC_BLK, TAIL_W), lambda i: (i, 0)),
        ],
        out_specs=pl.BlockSpec((TC_BLK, OUT_D), lambda i: (i, 0)),
        out_shape=jax.ShapeDtypeStruct((N, OUT_D), jnp.float32),
    )(ptg, cont, tail8)


def kernel(seq_pitch_type, seq_cont, seq_swing_attempt, seq_swing_result,
           pt_table, sr_table):
    B, L = seq_pitch_type.shape
    N = B * L
    pt_idx2 = seq_pitch_type.reshape(N // IDX_W, IDX_W).astype(jnp.int32)
    sr_idxT = seq_swing_result.reshape(N // 2, 2).astype(jnp.int32).T
    att = seq_swing_attempt.reshape(N)
    cont = seq_cont.reshape(N, NUM_CONT)
    sr8 = jnp.pad(sr_table, ((0, 0), (TAIL_W - SR_D, 0)))
    pair_tab = jnp.concatenate(
        [jnp.repeat(sr8, NUM_SR + 1, axis=0),
         jnp.tile(sr8, (NUM_SR + 1, 1))], axis=1)
    ptg, tail_pairs = _sc_gather(N, pt_idx2, sr_idxT, att, pt_table, pair_tab)
    tail8 = tail_pairs.reshape(N, TAIL_W)
    out = _tc_concat(N, ptg, cont, tail8)
    return out.reshape(B, L, OUT_D)
